# Initial kernel scaffold; baseline (speedup 1.0000x reference)
import jax
import jax.numpy as jnp
from jax.experimental import pallas as pl


def kernel(x, edge_index, batch, W0, b0, gamma0, beta0, W1, b1, gamma1, beta1, W2, b2, gamma2, beta2, W3, b3, gamma3, beta3, Wp, bp, Wo, bo):
    raise NotImplementedError("write your pallas kernel here")


# speedup vs baseline: 16.2636x; 16.2636x over previous
import functools

import jax
import jax.numpy as jnp
from jax import lax
from jax.experimental import pallas as pl
from jax.experimental.pallas import tpu as pltpu
from jax.experimental.pallas import tpu_sc as plsc

N = 10000
E = 640000
H = 128
G = 32
EPS = 1e-5

NC = 2
NS = 16
NW = NC * NS
LPR = 128
RPW = 160
RP = NW * RPW
EP = RP * LPR
NP = 10240
NPT = NP // NS
NPAD = NP - N
ICH = 8

_HIGH = jax.lax.Precision.HIGHEST


def _dot(a, b):
    return lax.dot_general(a, b, (((a.ndim - 1,), (0,)), ((), ())),
                           precision=_HIGH, preferred_element_type=jnp.float32)


_MESH = plsc.VectorSubcoreMesh(core_axis_name="c", subcore_axis_name="s")


@functools.partial(
    pl.kernel,
    out_type=jax.ShapeDtypeStruct((NC, NP, H), jnp.float32),
    mesh=_MESH,
    scratch_types=[
        pltpu.VMEM((ICH, LPR), jnp.int32),
        pltpu.VMEM((ICH, LPR), jnp.int32),
        pltpu.VMEM((LPR, H), jnp.float32),
        pltpu.VMEM_SHARED((NP, H), jnp.float32),
        pltpu.SemaphoreType.DMA,
    ],
)
def _spmm(u_hbm, src_hbm, dst_hbm, zero_hbm, out_hbm,
          src_v, dst_v, rows_v, acc, sem):
    c = lax.axis_index("c")
    s = lax.axis_index("s")
    wid = s * NC + c
    base = wid * RPW

    pltpu.sync_copy(zero_hbm, acc.at[pl.ds(s * NPT, NPT)])
    plsc.subcore_barrier()

    def chunk(k, carry):
        pltpu.sync_copy(src_hbm.at[pl.ds(base + k * ICH, ICH)], src_v)
        pltpu.sync_copy(dst_hbm.at[pl.ds(base + k * ICH, ICH)], dst_v)

        def body(i, c2):
            pltpu.async_copy(u_hbm.at[src_v.at[i]], rows_v, sem).wait()
            pltpu.sync_copy(rows_v, acc.at[dst_v.at[i]], add=True)
            return c2

        lax.fori_loop(0, ICH, body, 0)
        return carry

    lax.fori_loop(0, RPW // ICH, chunk, 0)
    plsc.subcore_barrier()
    pltpu.sync_copy(acc.at[pl.ds(s * NPT, NPT)],
                    out_hbm.at[c].at[pl.ds(s * NPT, NPT)])


@functools.partial(
    pl.kernel,
    out_type=jax.ShapeDtypeStruct((NC, NP, H), jnp.float32),
    mesh=_MESH,
    scratch_types=[
        pltpu.VMEM((ICH, LPR), jnp.int32),
        pltpu.VMEM((LPR, H), jnp.float32),
        pltpu.VMEM_SHARED((NP, H), jnp.float32),
    ],
)
def _deg(dst_hbm, zero_hbm, ones_hbm, out_hbm, dst_v, ones_v, acc):
    c = lax.axis_index("c")
    s = lax.axis_index("s")
    wid = s * NC + c
    base = wid * RPW

    pltpu.sync_copy(zero_hbm, acc.at[pl.ds(s * NPT, NPT)])
    pltpu.sync_copy(ones_hbm, ones_v)
    plsc.subcore_barrier()

    def chunk(k, carry):
        pltpu.sync_copy(dst_hbm.at[pl.ds(base + k * ICH, ICH)], dst_v)

        def body(i, c2):
            pltpu.sync_copy(ones_v, acc.at[dst_v.at[i]], add=True)
            return c2

        lax.fori_loop(0, ICH, body, 0)
        return carry

    lax.fori_loop(0, RPW // ICH, chunk, 0)
    plsc.subcore_barrier()
    pltpu.sync_copy(acc.at[pl.ds(s * NPT, NPT)],
                    out_hbm.at[c].at[pl.ds(s * NPT, NPT)])


def _tc0_body(degp_ref, x_ref, w_ref, b_ref, dinv_ref, u_ref):
    deg = degp_ref[0, 0:N, 0:1] + degp_ref[1, 0:N, 0:1] + 1.0
    dinv = lax.rsqrt(jnp.maximum(deg, 1.0))
    dinv_ref[...] = dinv
    u_ref[0:N, :] = dinv * (_dot(x_ref[...], w_ref[...]) + b_ref[...])
    u_ref[N:NP, :] = jnp.zeros((NPAD, H), jnp.float32)


_tc0 = pl.pallas_call(
    _tc0_body,
    out_shape=(jax.ShapeDtypeStruct((N, 1), jnp.float32),
               jax.ShapeDtypeStruct((NP, H), jnp.float32)),
)


def _bn_relu(p_ref, u_ref, dinv_ref, g_ref, bt_ref):
    dinv = dinv_ref[...]
    y = dinv * (p_ref[0, 0:N, :] + p_ref[1, 0:N, :] + u_ref[0:N, :])
    m = jnp.mean(y, axis=0, keepdims=True)
    v = jnp.mean((y - m) ** 2, axis=0, keepdims=True)
    return jnp.maximum((y - m) * lax.rsqrt(v + EPS) * g_ref[...] + bt_ref[...],
                       0.0), dinv


def _tc_mid_body(p_ref, u_ref, dinv_ref, g_ref, bt_ref, w_ref, b_ref,
                 unext_ref):
    h, dinv = _bn_relu(p_ref, u_ref, dinv_ref, g_ref, bt_ref)
    unext_ref[0:N, :] = dinv * (_dot(h, w_ref[...]) + b_ref[...])
    unext_ref[N:NP, :] = jnp.zeros((NPAD, H), jnp.float32)


_tc_mid = pl.pallas_call(
    _tc_mid_body,
    out_shape=jax.ShapeDtypeStruct((NP, H), jnp.float32),
)


def _tc_fin_body(p_ref, u_ref, dinv_ref, g_ref, bt_ref, batch_ref,
                 wp_ref, bp_ref, wo_ref, bo_ref, out_ref, mx_ref):
    h, _ = _bn_relu(p_ref, u_ref, dinv_ref, g_ref, bt_ref)
    gids = lax.broadcasted_iota(jnp.int32, (1, G), 1)
    oh = (batch_ref[...] == gids).astype(jnp.float32)
    cnt = jnp.sum(oh, axis=0, keepdims=True)
    seg_sum = lax.dot_general(oh, h, (((0,), (0,)), ((), ())),
                              precision=_HIGH,
                              preferred_element_type=jnp.float32)
    seg_mean = seg_sum / jnp.maximum(cnt, 1.0).reshape(G, 1)

    def body(g, carry):
        mask = batch_ref[...] == g
        mx_ref[pl.ds(g, 1), :] = jnp.max(jnp.where(mask, h, 0.0), axis=0,
                                         keepdims=True)
        return carry

    lax.fori_loop(0, G, body, 0)
    pooled = jnp.concatenate([seg_mean, mx_ref[...]], axis=1)
    hid = jnp.maximum(_dot(pooled, wp_ref[...]) + bp_ref[...], 0.0)
    out_ref[...] = _dot(hid, wo_ref[...]) + bo_ref[...]


_tc_fin = pl.pallas_call(
    _tc_fin_body,
    out_shape=jax.ShapeDtypeStruct((G, H // 2), jnp.float32),
    scratch_shapes=[pltpu.VMEM((G, H), jnp.float32)],
)


def _pad_edges(edge_index):
    pad = N + (jnp.arange(EP - E, dtype=jnp.int32) % NPAD)
    src2 = jnp.concatenate([edge_index[0], pad]).reshape(RP, LPR)
    dst2 = jnp.concatenate([edge_index[1], pad]).reshape(RP, LPR)
    return src2, dst2


def kernel(x, edge_index, batch, W0, b0, gamma0, beta0, W1, b1, gamma1, beta1,
           W2, b2, gamma2, beta2, W3, b3, gamma3, beta3, Wp, bp, Wo, bo):
    src2, dst2 = _pad_edges(edge_index)
    zero_h = jnp.zeros((NPT, H), jnp.float32)

    degp = _deg(dst2, zero_h, jnp.ones((LPR, H), jnp.float32))
    dinv, u = _tc0(degp, x, W0, b0.reshape(1, H))

    layer = [(gamma0, beta0, W1, b1), (gamma1, beta1, W2, b2),
             (gamma2, beta2, W3, b3)]
    for g, bt, w, b in layer:
        p = _spmm(u, src2, dst2, zero_h)
        u = _tc_mid(p, u, dinv, g.reshape(1, H), bt.reshape(1, H), w,
                    b.reshape(1, H))

    p = _spmm(u, src2, dst2, zero_h)
    return _tc_fin(p, u, dinv, gamma3.reshape(1, H), beta3.reshape(1, H),
                   batch.reshape(N, 1), Wp, bp.reshape(1, H), Wo,
                   bo.reshape(1, H // 2))


# speedup vs baseline: 23.8577x; 1.4669x over previous
import functools

import jax
import jax.numpy as jnp
from jax import lax
from jax.experimental import pallas as pl
from jax.experimental.pallas import tpu as pltpu
from jax.experimental.pallas import tpu_sc as plsc

N = 10000
E = 640000
H = 128
G = 32
EPS = 1e-5

NC = 2
NS = 16
NW = NC * NS
LPR = 128
RPW = 160
RP = NW * RPW
EP = RP * LPR
NP = 10240
NPT = NP // NS
NPAD = NP - N
ICH = 16

_HIGH = jax.lax.Precision.HIGHEST


def _dot(a, b):
    return lax.dot_general(a, b, (((a.ndim - 1,), (0,)), ((), ())),
                           precision=_HIGH, preferred_element_type=jnp.float32)


_MESH = plsc.VectorSubcoreMesh(core_axis_name="c", subcore_axis_name="s")


@functools.partial(
    pl.kernel,
    out_type=jax.ShapeDtypeStruct((NC, NP, H), jnp.float32),
    mesh=_MESH,
    scratch_types=[
        pltpu.VMEM((ICH, LPR), jnp.int32),
        pltpu.VMEM((ICH, LPR), jnp.int32),
        pltpu.VMEM((LPR, H), jnp.float32),
        pltpu.VMEM((LPR, H), jnp.float32),
        pltpu.VMEM_SHARED((NP, H), jnp.float32),
        pltpu.SemaphoreType.DMA,
        pltpu.SemaphoreType.DMA,
        pltpu.SemaphoreType.DMA,
        pltpu.SemaphoreType.DMA,
    ],
)
def _spmm(u_hbm, src_hbm, dst_hbm, zero_hbm, out_hbm,
          src_v, dst_v, rows_a, rows_b, acc, gsa, gsb, ssa, ssb):
    c = lax.axis_index("c")
    s = lax.axis_index("s")
    wid = s * NC + c
    base = wid * RPW

    pltpu.sync_copy(zero_hbm, acc.at[pl.ds(s * NPT, NPT)])
    plsc.subcore_barrier()

    bufs = (rows_a, rows_b)
    gsems = (gsa, gsb)
    ssems = (ssa, ssb)

    def chunk(k, carry):
        pltpu.sync_copy(src_hbm.at[pl.ds(base + k * ICH, ICH)], src_v)
        pltpu.sync_copy(dst_hbm.at[pl.ds(base + k * ICH, ICH)], dst_v)
        g = [None, None]
        sc = [None, None]
        for i in range(ICH):
            b = i % 2
            if sc[b] is not None:
                sc[b].wait()
            g[b] = pltpu.async_copy(u_hbm.at[src_v.at[i]], bufs[b], gsems[b])
            if i >= 1:
                pb = (i - 1) % 2
                g[pb].wait()
                sc[pb] = pltpu.async_copy(bufs[pb],
                                          acc.at[dst_v.at[i - 1]],
                                          ssems[pb], add=True)
        lb = (ICH - 1) % 2
        g[lb].wait()
        sc[lb] = pltpu.async_copy(bufs[lb], acc.at[dst_v.at[ICH - 1]],
                                  ssems[lb], add=True)
        sc[0].wait()
        sc[1].wait()
        return carry

    lax.fori_loop(0, RPW // ICH, chunk, 0)
    plsc.subcore_barrier()
    pltpu.sync_copy(acc.at[pl.ds(s * NPT, NPT)],
                    out_hbm.at[c].at[pl.ds(s * NPT, NPT)])


@functools.partial(
    pl.kernel,
    out_type=jax.ShapeDtypeStruct((NC, NP, H), jnp.float32),
    mesh=_MESH,
    scratch_types=[
        pltpu.VMEM((ICH, LPR), jnp.int32),
        pltpu.VMEM((LPR, H), jnp.float32),
        pltpu.VMEM_SHARED((NP, H), jnp.float32),
        pltpu.SemaphoreType.DMA,
    ],
)
def _deg(dst_hbm, zero_hbm, ones_hbm, out_hbm, dst_v, ones_v, acc, sem):
    c = lax.axis_index("c")
    s = lax.axis_index("s")
    wid = s * NC + c
    base = wid * RPW

    pltpu.sync_copy(zero_hbm, acc.at[pl.ds(s * NPT, NPT)])
    pltpu.sync_copy(ones_hbm, ones_v)
    plsc.subcore_barrier()

    def chunk(k, carry):
        pltpu.sync_copy(dst_hbm.at[pl.ds(base + k * ICH, ICH)], dst_v)
        descs = [pltpu.async_copy(ones_v, acc.at[dst_v.at[i]], sem, add=True)
                 for i in range(ICH)]
        for d in descs:
            d.wait()
        return carry

    lax.fori_loop(0, RPW // ICH, chunk, 0)
    plsc.subcore_barrier()
    pltpu.sync_copy(acc.at[pl.ds(s * NPT, NPT)],
                    out_hbm.at[c].at[pl.ds(s * NPT, NPT)])


def _tc0_body(degp_ref, x_ref, w_ref, b_ref, dinv_ref, u_ref):
    deg = degp_ref[0, 0:N, 0:1] + degp_ref[1, 0:N, 0:1] + 1.0
    dinv = lax.rsqrt(jnp.maximum(deg, 1.0))
    dinv_ref[...] = dinv
    u_ref[0:N, :] = dinv * (_dot(x_ref[...], w_ref[...]) + b_ref[...])
    u_ref[N:NP, :] = jnp.zeros((NPAD, H), jnp.float32)


_tc0 = pl.pallas_call(
    _tc0_body,
    out_shape=(jax.ShapeDtypeStruct((N, 1), jnp.float32),
               jax.ShapeDtypeStruct((NP, H), jnp.float32)),
)


def _bn_relu(p_ref, u_ref, dinv_ref, g_ref, bt_ref):
    dinv = dinv_ref[...]
    y = dinv * (p_ref[0, 0:N, :] + p_ref[1, 0:N, :] + u_ref[0:N, :])
    m = jnp.mean(y, axis=0, keepdims=True)
    v = jnp.mean((y - m) ** 2, axis=0, keepdims=True)
    return jnp.maximum((y - m) * lax.rsqrt(v + EPS) * g_ref[...] + bt_ref[...],
                       0.0), dinv


def _tc_mid_body(p_ref, u_ref, dinv_ref, g_ref, bt_ref, w_ref, b_ref,
                 unext_ref):
    h, dinv = _bn_relu(p_ref, u_ref, dinv_ref, g_ref, bt_ref)
    unext_ref[0:N, :] = dinv * (_dot(h, w_ref[...]) + b_ref[...])
    unext_ref[N:NP, :] = jnp.zeros((NPAD, H), jnp.float32)


_tc_mid = pl.pallas_call(
    _tc_mid_body,
    out_shape=jax.ShapeDtypeStruct((NP, H), jnp.float32),
)


def _tc_fin_body(p_ref, u_ref, dinv_ref, g_ref, bt_ref, batch_ref,
                 wp_ref, bp_ref, wo_ref, bo_ref, out_ref, mx_ref):
    h, _ = _bn_relu(p_ref, u_ref, dinv_ref, g_ref, bt_ref)
    gids = lax.broadcasted_iota(jnp.int32, (1, G), 1)
    oh = (batch_ref[...] == gids).astype(jnp.float32)
    cnt = jnp.sum(oh, axis=0, keepdims=True)
    seg_sum = lax.dot_general(oh, h, (((0,), (0,)), ((), ())),
                              precision=_HIGH,
                              preferred_element_type=jnp.float32)
    seg_mean = seg_sum / jnp.maximum(cnt, 1.0).reshape(G, 1)

    def body(g, carry):
        mask = batch_ref[...] == g
        mx_ref[pl.ds(g, 1), :] = jnp.max(jnp.where(mask, h, 0.0), axis=0,
                                         keepdims=True)
        return carry

    lax.fori_loop(0, G, body, 0)
    pooled = jnp.concatenate([seg_mean, mx_ref[...]], axis=1)
    hid = jnp.maximum(_dot(pooled, wp_ref[...]) + bp_ref[...], 0.0)
    out_ref[...] = _dot(hid, wo_ref[...]) + bo_ref[...]


_tc_fin = pl.pallas_call(
    _tc_fin_body,
    out_shape=jax.ShapeDtypeStruct((G, H // 2), jnp.float32),
    scratch_shapes=[pltpu.VMEM((G, H), jnp.float32)],
)


def _pad_edges(edge_index):
    pad = N + (jnp.arange(EP - E, dtype=jnp.int32) % NPAD)
    src2 = jnp.concatenate([edge_index[0], pad]).reshape(RP, LPR)
    dst2 = jnp.concatenate([edge_index[1], pad]).reshape(RP, LPR)
    return src2, dst2


def kernel(x, edge_index, batch, W0, b0, gamma0, beta0, W1, b1, gamma1, beta1,
           W2, b2, gamma2, beta2, W3, b3, gamma3, beta3, Wp, bp, Wo, bo):
    src2, dst2 = _pad_edges(edge_index)
    zero_h = jnp.zeros((NPT, H), jnp.float32)

    degp = _deg(dst2, zero_h, jnp.ones((LPR, H), jnp.float32))
    dinv, u = _tc0(degp, x, W0, b0.reshape(1, H))

    layer = [(gamma0, beta0, W1, b1), (gamma1, beta1, W2, b2),
             (gamma2, beta2, W3, b3)]
    for g, bt, w, b in layer:
        p = _spmm(u, src2, dst2, zero_h)
        u = _tc_mid(p, u, dinv, g.reshape(1, H), bt.reshape(1, H), w,
                    b.reshape(1, H))

    p = _spmm(u, src2, dst2, zero_h)
    return _tc_fin(p, u, dinv, gamma3.reshape(1, H), beta3.reshape(1, H),
                   batch.reshape(N, 1), Wp, bp.reshape(1, H), Wo,
                   bo.reshape(1, H // 2))
